# baseline (speedup 1.0000x reference)
import functools

import jax
import jax.numpy as jnp
from jax import lax
from jax.experimental import pallas as pl
from jax.experimental.pallas import tpu as pltpu
from jax.experimental.pallas import tpu_sc as plsc

_B = 16384
_K = 20
_D = 64
_NC = 2
_NS = 16
_NW = _NC * _NS
_BPW = _B // _NW
_T = 32
_NT = _BPW // _T
_TK = _T * _K
_NCHUNK = _TK // 128
_OC = 32

_H = 500224
_PW = 512
_PG = _H // _PW
_NROWS = 2 * _H


def _pack_body(a_ref, b_ref, o_ref):
    ii = lax.broadcasted_iota(jnp.int32, (_D, _D), 0)
    jj = lax.broadcasted_iota(jnp.int32, (_D, _D), 1)
    eye = jnp.where(ii == jj, 1.0, 0.0).astype(jnp.float32)
    dn = (((0,), (0,)), ((), ()))
    ya = lax.dot_general(a_ref[...], eye, dn,
                         preferred_element_type=jnp.float32)
    yb = lax.dot_general(b_ref[...], eye, dn,
                         preferred_element_type=jnp.float32)
    o_ref[...] = jnp.concatenate([ya, yb], axis=1)


def _pack(tT):
    return pl.pallas_call(
        _pack_body,
        grid=(_PG,),
        in_specs=[
            pl.BlockSpec((_D, _PW), lambda j: (0, j)),
            pl.BlockSpec((_D, _PW), lambda j: (0, j + _PG)),
        ],
        out_specs=pl.BlockSpec((_PW, 2 * _D), lambda j: (j, 0)),
        out_shape=jax.ShapeDtypeStruct((_H, 2 * _D), jnp.float32),
    )(tT, tT)


def _remap(idx16):
    hi = (idx16 >= _H).astype(jnp.int32)
    return idx16 + idx16 - hi * (2 * _H - 1)


def _sc_logits_kernel(src_hbm, pos_hbm, negs_hbm, emb_hbm, ctx_hbm, out_hbm,
                      sidx0, sidx1, pidx0, pidx1, nidx0, nidx1,
                      erows0, erows1, prows0, prows1, nrows0, nrows1,
                      outv, scr, sem0, sem1):
    wid = lax.axis_index("s") * _NC + lax.axis_index("c")
    base = wid * _BPW
    col0 = lax.iota(jnp.int32, 16) * 16
    bufs = ((sidx0, pidx0, nidx0, erows0, prows0, nrows0, sem0),
            (sidx1, pidx1, nidx1, erows1, prows1, nrows1, sem1))

    def gather_descs(b):
        sidx, pidx, nidx, erows, prows, nrows, sem = bufs[b]
        descs = [(emb_hbm.at[sidx], erows, sem), (ctx_hbm.at[pidx], prows, sem)]
        for c in range(_NCHUNK):
            descs.append((ctx_hbm.at[nidx.at[pl.ds(c * 128, 128)]],
                          nrows.at[pl.ds(c * 128, 128)], sem))
        return descs

    def issue(t, b):
        sidx, pidx, nidx, erows, prows, nrows, sem = bufs[b]
        tb = base + t * _T
        pltpu.sync_copy(src_hbm.at[pl.ds(tb, _T)], sidx)
        pltpu.sync_copy(pos_hbm.at[pl.ds(tb, _T)], pidx)
        pltpu.sync_copy(negs_hbm.at[pl.ds(tb * _K, _TK)], nidx)
        for c in range(_T // 16):
            sidx[pl.ds(c * 16, 16)] = _remap(sidx[pl.ds(c * 16, 16)])
            pidx[pl.ds(c * 16, 16)] = _remap(pidx[pl.ds(c * 16, 16)])
        for c in range(_TK // 16):
            nidx[pl.ds(c * 16, 16)] = _remap(nidx[pl.ds(c * 16, 16)])
        for d in gather_descs(b):
            pltpu.async_copy(*d)

    def drain(b):
        for d in gather_descs(b):
            pltpu.make_async_copy(*d).wait()

    def compute(t, b):
        _, _, _, erows, prows, nrows, _ = bufs[b]
        tb = base + t * _T

        def sample_body(i, carry2):
            e = [erows[i, pl.ds(c * 16, 16)] for c in range(4)]

            def dot_to(j, load_r):
                p01 = e[0] * load_r(0) + e[1] * load_r(1)
                p23 = e[2] * load_r(2) + e[3] * load_r(3)
                scr[pl.ds(j * 16, 16)] = p01 + p23

            def col_sum():
                acc = plsc.load_gather(scr, [col0])
                for c in range(1, 16):
                    acc = acc + plsc.load_gather(scr, [col0 + c])
                return acc

            dot_to(0, lambda c: prows[i, pl.ds(c * 16, 16)])
            for k in range(15):
                dot_to(1 + k,
                       lambda c, k=k: nrows[i * _K + k, pl.ds(c * 16, 16)])
            outv[i, pl.ds(0, 16)] = col_sum()
            for k in range(15, _K):
                dot_to(k - 15,
                       lambda c, k=k: nrows[i * _K + k, pl.ds(c * 16, 16)])
            outv[i, pl.ds(16, 16)] = col_sum()
            return carry2

        lax.fori_loop(0, _T, sample_body, 0)
        pltpu.sync_copy(outv, out_hbm.at[pl.ds(tb, _T), :])

    issue(0, 0)

    def pair_body(g, carry):
        t0 = 2 * g
        issue(t0 + 1, 1)
        drain(0)
        compute(t0, 0)

        @pl.when(t0 + 2 < _NT)
        def _():
            issue(t0 + 2, 0)

        drain(1)
        compute(t0 + 1, 1)
        return carry

    lax.fori_loop(0, _NT // 2, pair_body, 0)


_sc_call = pl.kernel(
    _sc_logits_kernel,
    out_type=jax.ShapeDtypeStruct((_B, _OC), jnp.float32),
    mesh=plsc.VectorSubcoreMesh(core_axis_name="c", subcore_axis_name="s"),
    compiler_params=pltpu.CompilerParams(needs_layout_passes=False,
                                         use_tc_tiling_on_sc=False),
    scratch_types=[
        pltpu.VMEM((_T,), jnp.int32),
        pltpu.VMEM((_T,), jnp.int32),
        pltpu.VMEM((_T,), jnp.int32),
        pltpu.VMEM((_T,), jnp.int32),
        pltpu.VMEM((_TK,), jnp.int32),
        pltpu.VMEM((_TK,), jnp.int32),
        pltpu.VMEM((_T, _D), jnp.float32),
        pltpu.VMEM((_T, _D), jnp.float32),
        pltpu.VMEM((_T, _D), jnp.float32),
        pltpu.VMEM((_T, _D), jnp.float32),
        pltpu.VMEM((_TK, _D), jnp.float32),
        pltpu.VMEM((_TK, _D), jnp.float32),
        pltpu.VMEM((_T, _OC), jnp.float32),
        pltpu.VMEM((256,), jnp.float32),
        pltpu.SemaphoreType.DMA,
        pltpu.SemaphoreType.DMA,
    ],
)


def _tc_loss_kernel(x_ref, o_ref):
    x = x_ref[...]
    col = lax.broadcasted_iota(jnp.int32, x.shape, 1) % _OC
    z = jnp.where(col == 0, x, -x)
    ls = jnp.minimum(z, 0.0) - jnp.log1p(jnp.exp(-jnp.abs(z)))
    ls = jnp.where(col < _K + 1, ls, 0.0)
    o_ref[0, 0] = -jnp.sum(ls) / _B


@jax.jit
def kernel(src, pos, negs, embedder_W, context_W):
    emb_l = _pack(embedder_W.T).reshape(_NROWS, _D)
    ctx_l = _pack(context_W.T).reshape(_NROWS, _D)
    logits = _sc_call(src.reshape(_B), pos.reshape(_B),
                      negs.reshape(_B * _K),
                      emb_l, ctx_l)
    x = logits.reshape(_B * _OC // 128, 128)
    loss = pl.pallas_call(
        _tc_loss_kernel,
        out_shape=jax.ShapeDtypeStruct((1, 1), jnp.float32),
        out_specs=pl.BlockSpec(memory_space=pltpu.SMEM),
    )(x)
    return loss[0, 0]


# speedup vs baseline: 1.3126x; 1.3126x over previous
import jax
import jax.numpy as jnp
from jax import lax
from jax.experimental import pallas as pl
from jax.experimental.pallas import tpu as pltpu
from jax.experimental.pallas import tpu_sc as plsc

_B = 16384
_K = 20
_D = 64
_NC = 2
_NS = 16
_NW = _NC * _NS
_BPW = _B // _NW
_T = 32
_NT = _BPW // _T
_TK = _T * _K
_NCHUNK = _TK // 128
_OC = 32


def _sc_logits_kernel(src_hbm, pos_hbm, negs_hbm, emb_hbm, ctx_hbm, out_hbm,
                      sidx, pidx, nidx,
                      erows0, erows1, prows0, prows1, nrows0, nrows1,
                      outv, scr, sem0, sem1):
    wid = lax.axis_index("s") * _NC + lax.axis_index("c")
    base = wid * _BPW
    col0 = lax.iota(jnp.int32, 16) * 16
    bufs = ((erows0, prows0, nrows0, sem0),
            (erows1, prows1, nrows1, sem1))

    pltpu.sync_copy(src_hbm.at[pl.ds(base, _BPW)], sidx)
    pltpu.sync_copy(pos_hbm.at[pl.ds(base, _BPW)], pidx)
    pltpu.sync_copy(negs_hbm.at[pl.ds(base * _K, _BPW * _K)], nidx)

    def gather_descs(t, b):
        erows, prows, nrows, sem = bufs[b]
        descs = [(emb_hbm.at[sidx.at[pl.ds(t * _T, _T)]], erows, sem),
                 (ctx_hbm.at[pidx.at[pl.ds(t * _T, _T)]], prows, sem)]
        for c in range(_NCHUNK):
            descs.append(
                (ctx_hbm.at[nidx.at[pl.ds(t * _TK + c * 128, 128)]],
                 nrows.at[pl.ds(c * 128, 128)], sem))
        return descs

    def issue(t, b):
        for d in gather_descs(t, b):
            pltpu.async_copy(*d)

    def drain(t, b):
        for d in gather_descs(t, b):
            pltpu.make_async_copy(*d).wait()

    def compute(t, b):
        erows, prows, nrows, _ = bufs[b]
        tb = base + t * _T

        def sample_body(i, carry2):
            e = [erows[i, pl.ds(c * 16, 16)] for c in range(4)]

            def dot_to(j, load_r):
                p01 = e[0] * load_r(0) + e[1] * load_r(1)
                p23 = e[2] * load_r(2) + e[3] * load_r(3)
                scr[pl.ds(j * 16, 16)] = p01 + p23

            def col_sum():
                acc = plsc.load_gather(scr, [col0])
                for c in range(1, 16):
                    acc = acc + plsc.load_gather(scr, [col0 + c])
                return acc

            dot_to(0, lambda c: prows[i, pl.ds(c * 16, 16)])
            for k in range(15):
                dot_to(1 + k,
                       lambda c, k=k: nrows[i * _K + k, pl.ds(c * 16, 16)])
            outv[i, pl.ds(0, 16)] = col_sum()
            for k in range(15, _K):
                dot_to(k - 15,
                       lambda c, k=k: nrows[i * _K + k, pl.ds(c * 16, 16)])
            outv[i, pl.ds(16, 16)] = col_sum()
            return carry2

        lax.fori_loop(0, _T, sample_body, 0)
        pltpu.sync_copy(outv, out_hbm.at[pl.ds(tb, _T), :])

    issue(0, 0)

    def pair_body(g, carry):
        t0 = 2 * g
        issue(t0 + 1, 1)
        drain(t0, 0)
        compute(t0, 0)

        @pl.when(t0 + 2 < _NT)
        def _():
            issue(t0 + 2, 0)

        drain(t0 + 1, 1)
        compute(t0 + 1, 1)
        return carry

    lax.fori_loop(0, _NT // 2, pair_body, 0)


_sc_call = pl.kernel(
    _sc_logits_kernel,
    out_type=jax.ShapeDtypeStruct((_B, _OC), jnp.float32),
    mesh=plsc.VectorSubcoreMesh(core_axis_name="c", subcore_axis_name="s"),
    compiler_params=pltpu.CompilerParams(needs_layout_passes=False,
                                         use_tc_tiling_on_sc=False),
    scratch_types=[
        pltpu.VMEM((_BPW,), jnp.int32),
        pltpu.VMEM((_BPW,), jnp.int32),
        pltpu.VMEM((_BPW * _K,), jnp.int32),
        pltpu.VMEM((_T, _D), jnp.float32),
        pltpu.VMEM((_T, _D), jnp.float32),
        pltpu.VMEM((_T, _D), jnp.float32),
        pltpu.VMEM((_T, _D), jnp.float32),
        pltpu.VMEM((_TK, _D), jnp.float32),
        pltpu.VMEM((_TK, _D), jnp.float32),
        pltpu.VMEM((_T, _OC), jnp.float32),
        pltpu.VMEM((256,), jnp.float32),
        pltpu.SemaphoreType.DMA,
        pltpu.SemaphoreType.DMA,
    ],
)


def _tc_loss_kernel(x_ref, o_ref):
    x = x_ref[...]
    col = lax.broadcasted_iota(jnp.int32, x.shape, 1) % _OC
    z = jnp.where(col == 0, x, -x)
    ls = jnp.minimum(z, 0.0) - jnp.log1p(jnp.exp(-jnp.abs(z)))
    ls = jnp.where(col < _K + 1, ls, 0.0)
    o_ref[0, 0] = -jnp.sum(ls) / _B


@jax.jit
def kernel(src, pos, negs, embedder_W, context_W):
    logits = _sc_call(src.reshape(_B), pos.reshape(_B),
                      negs.reshape(_B * _K),
                      embedder_W, context_W)
    x = logits.reshape(_B * _OC // 128, 128)
    loss = pl.pallas_call(
        _tc_loss_kernel,
        out_shape=jax.ShapeDtypeStruct((1, 1), jnp.float32),
        out_specs=pl.BlockSpec(memory_space=pltpu.SMEM),
    )(x)
    return loss[0, 0]
